# baseline (speedup 1.0000x reference)
import functools

import jax
import jax.numpy as jnp
from jax import lax
from jax.experimental import pallas as pl
from jax.experimental.pallas import tpu as pltpu
from jax.experimental.pallas import tpu_sc as plsc

IN_DIM = 4096
OUT_DIM = 4096
BATCH = 1024
W_WORDS = IN_DIM * OUT_DIM

NC = 2
NS = 16
CHUNK = 2048
NCHUNK = 52
TW_IN = NCHUNK * CHUNK
NNZ_PAD = NS * TW_IN
REG = 1703936
TW_REG = REG // NS
NROUNDS = 5
ROWS_PER_REG = REG // OUT_DIM
ZCHUNK = 2048
NZCOPY = TW_REG // ZCHUNK
JROWS = CHUNK // 128

_mesh = plsc.VectorSubcoreMesh(
    core_axis_name="c", subcore_axis_name="s", num_cores=NC, num_subcores=NS
)

_SCRATCH = [
    pltpu.VMEM((2, CHUNK), jnp.int32),
    pltpu.VMEM((2, CHUNK), jnp.int32),
    pltpu.VMEM((2, CHUNK), jnp.float32),
    pltpu.VMEM((2, JROWS, 128), jnp.int32),
    pltpu.VMEM((2, CHUNK), jnp.float32),
    pltpu.VMEM((ZCHUNK,), jnp.float32),
    pltpu.VMEM_SHARED((REG,), jnp.float32),
    pltpu.SemaphoreType.DMA,
    pltpu.SemaphoreType.DMA,
    pltpu.SemaphoreType.DMA,
]


def _make_scatter_round(rnd):
    out_words = 2 * REG if rnd < NROUNDS - 1 else W_WORDS - 8 * REG

    @functools.partial(
        pl.kernel,
        out_type=jax.ShapeDtypeStruct((out_words,), jnp.float32),
        mesh=_mesh,
        scratch_types=_SCRATCH,
        name=f"scatter_round{rnd}",
    )
    def _scatter(rows_hbm, cols_hbm, vals_hbm, w_hbm,
                 rbuf, cbuf, vbuf, idxs, vstage, zeros, acc,
                 lsem, ssem, zsem):
        c = lax.axis_index("c")
        s = lax.axis_index("s")
        in_base = s * TW_IN
        base = (rnd * NC + c) * REG

        zvec = jnp.zeros((16,), jnp.float32)

        def _zinit(i, _):
            zeros[pl.ds(pl.multiple_of(i * 16, 16), 16)] = zvec
            return ()

        lax.fori_loop(0, ZCHUNK // 16, _zinit, ())

        sbase = lax.iota(jnp.int32, 16) * 8192 + s * 512

        def _issue_loads(k, par):
            off = pl.multiple_of(in_base + k * CHUNK, CHUNK)
            pltpu.async_copy(
                rows_hbm.at[pl.ds(off, CHUNK)], rbuf.at[par], lsem)
            pltpu.async_copy(
                cols_hbm.at[pl.ds(off, CHUNK)], cbuf.at[par], lsem)
            pltpu.async_copy(
                vals_hbm.at[pl.ds(off, CHUNK)], vbuf.at[par], lsem)

        def _drain_loads():
            for ref in (rbuf, cbuf, vbuf):
                pltpu.make_async_copy(
                    rows_hbm.at[pl.ds(0, CHUNK)], ref.at[0], lsem
                ).wait()

        def _drain_streams(n):
            for _ in range(n):
                pltpu.make_async_copy(
                    vstage.at[0, pl.ds(0, 128)], acc.at[idxs.at[0, 0]], ssem
                ).wait()

        zh = [
            pltpu.async_copy(
                zeros, acc.at[pl.ds(s * TW_REG + i * ZCHUNK, ZCHUNK)], zsem
            )
            for i in range(NZCOPY)
        ]
        for h in zh:
            h.wait()
        plsc.subcore_barrier()

        _issue_loads(0, 0)

        def _chunk(k, _):
            par = lax.rem(k, 2)
            _drain_loads()

            @pl.when(k + 1 < NCHUNK)
            def _prefetch():
                _issue_loads(k + 1, 1 - par)

            @pl.when(k >= 2)
            def _older():
                _drain_streams(JROWS)

            for j in range(JROWS):
                def _vec(m, _):
                    o = pl.multiple_of(j * 128 + m * 16, 16)
                    sl = pl.ds(o, 16)
                    idx = cbuf[par, sl] * OUT_DIM + rbuf[par, sl] - base
                    valid = (idx >= 0) & (idx < REG)
                    idxs[par, j, pl.ds(pl.multiple_of(m * 16, 16), 16)] = (
                        jnp.where(valid, idx, sbase + (j * 128 + m * 16))
                    )
                    vstage[par, sl] = jnp.where(valid, vbuf[par, sl], 0.0)
                    return ()

                for m in range(8):
                    _vec(m, ())
                pltpu.async_copy(
                    vstage.at[par, pl.ds(j * 128, 128)],
                    acc.at[idxs.at[par, j]],
                    ssem,
                    add=True,
                )
            return ()

        lax.fori_loop(0, NCHUNK, _chunk, ())
        _drain_streams(2 * JROWS)
        plsc.subcore_barrier()

        src_full = acc.at[pl.ds(s * TW_REG, TW_REG)]
        dst_full = w_hbm.at[pl.ds(c * REG + s * TW_REG, TW_REG)]
        if rnd < NROUNDS - 1:
            pltpu.sync_copy(src_full, dst_full)
        else:
            @pl.when((c == 0) | (s < 13))
            def _full():
                pltpu.sync_copy(src_full, dst_full)

            @pl.when((c == 1) & (s == 13))
            def _partial():
                pltpu.sync_copy(
                    acc.at[pl.ds(13 * TW_REG, 57344)],
                    w_hbm.at[pl.ds(REG + 13 * TW_REG, 57344)],
                )

    return _scatter


_SCATTER_ROUNDS = [_make_scatter_round(r) for r in range(NROUNDS)]


BN = 1024


def _matmul_acc_body(x_ref, wt_ref, acc_ref, out_ref):
    out_ref[...] = acc_ref[...] + jnp.dot(
        x_ref[...],
        wt_ref[...].astype(jnp.bfloat16),
        preferred_element_type=jnp.float32,
    )


def _matmul_acc(x_r, wt_r, acc):
    kr = x_r.shape[1]
    return pl.pallas_call(
        _matmul_acc_body,
        grid=(OUT_DIM // BN,),
        in_specs=[
            pl.BlockSpec((BATCH, kr), lambda n: (0, 0)),
            pl.BlockSpec((kr, BN), lambda n: (0, n)),
            pl.BlockSpec((BATCH, BN), lambda n: (0, n)),
        ],
        out_specs=pl.BlockSpec((BATCH, BN), lambda n: (0, n)),
        out_shape=jax.ShapeDtypeStruct((BATCH, OUT_DIM), jnp.float32),
        input_output_aliases={2: 0},
        compiler_params=pltpu.CompilerParams(
            dimension_semantics=("arbitrary",),
        ),
    )(x_r, wt_r, acc)


def kernel(x, vals, b, rows, cols):
    pad = NNZ_PAD - rows.shape[0]
    t = jnp.arange(pad, dtype=jnp.int32)
    rows_p = jnp.concatenate([rows, (t * 37) % OUT_DIM])
    cols_p = jnp.concatenate([cols, t % IN_DIM])
    vals_p = jnp.concatenate([vals, jnp.zeros((pad,), jnp.float32)])

    xb = x.astype(jnp.bfloat16)
    acc = jnp.broadcast_to(b, (BATCH, OUT_DIM)) + jnp.zeros(
        (BATCH, OUT_DIM), jnp.float32
    )
    kr0 = 2 * ROWS_PER_REG
    for rnd in range(NROUNDS):
        wt_flat = _SCATTER_ROUNDS[rnd](rows_p, cols_p, vals_p)
        kr = wt_flat.shape[0] // OUT_DIM
        wt_r = wt_flat.reshape(kr, OUT_DIM)
        x_r = lax.slice_in_dim(xb, rnd * kr0, rnd * kr0 + kr, axis=1)
        acc = _matmul_acc(x_r, wt_r, acc)
    return acc


# speedup vs baseline: 1.2663x; 1.2663x over previous
import functools

import jax
import jax.numpy as jnp
from jax import lax
from jax.experimental import pallas as pl
from jax.experimental.pallas import tpu as pltpu
from jax.experimental.pallas import tpu_sc as plsc

IN_DIM = 4096
OUT_DIM = 4096
BATCH = 1024
W_WORDS = IN_DIM * OUT_DIM

NC = 2
NS = 16
CHUNK = 2048
NCHUNK = 52
TW_IN = NCHUNK * CHUNK
NNZ_PAD = NS * TW_IN
REG = 1703936
TW_REG = REG // NS
NROUNDS = 5
ROWS_PER_REG = REG // OUT_DIM
ZCHUNK = 2048
NZCOPY = TW_REG // ZCHUNK
JROWS = CHUNK // 128

_mesh = plsc.VectorSubcoreMesh(
    core_axis_name="c", subcore_axis_name="s", num_cores=NC, num_subcores=NS
)

_SCRATCH = [
    pltpu.VMEM((2, CHUNK), jnp.int32),
    pltpu.VMEM((2, CHUNK), jnp.int32),
    pltpu.VMEM((2, CHUNK), jnp.float32),
    pltpu.VMEM((2, JROWS, 128), jnp.int32),
    pltpu.VMEM((2, CHUNK), jnp.float32),
    pltpu.VMEM((ZCHUNK,), jnp.float32),
    pltpu.VMEM_SHARED((REG,), jnp.float32),
    pltpu.SemaphoreType.DMA,
    pltpu.SemaphoreType.DMA,
    pltpu.SemaphoreType.DMA,
]


def _make_scatter_round(rnd):
    out_words = 2 * REG if rnd < NROUNDS - 1 else W_WORDS - 8 * REG

    @functools.partial(
        pl.kernel,
        out_type=jax.ShapeDtypeStruct((out_words,), jnp.float32),
        mesh=_mesh,
        scratch_types=_SCRATCH,
        name=f"scatter_round{rnd}",
    )
    def _scatter(rows_hbm, cols_hbm, vals_hbm, w_hbm,
                 rbuf, cbuf, vbuf, idxs, vstage, zeros, acc,
                 lsem, ssem, zsem):
        c = lax.axis_index("c")
        s = lax.axis_index("s")
        in_base = s * TW_IN
        base = (rnd * NC + c) * REG

        zvec = jnp.zeros((16,), jnp.float32)

        def _zinit(i, _):
            zeros[pl.ds(pl.multiple_of(i * 16, 16), 16)] = zvec
            return ()

        lax.fori_loop(0, ZCHUNK // 16, _zinit, ())

        sbase = lax.iota(jnp.int32, 16) * 8192 + s * 512

        def _issue_loads(k, par):
            off = pl.multiple_of(in_base + k * CHUNK, CHUNK)
            pltpu.async_copy(
                rows_hbm.at[pl.ds(off, CHUNK)], rbuf.at[par], lsem)
            pltpu.async_copy(
                cols_hbm.at[pl.ds(off, CHUNK)], cbuf.at[par], lsem)
            pltpu.async_copy(
                vals_hbm.at[pl.ds(off, CHUNK)], vbuf.at[par], lsem)

        def _drain_loads():
            for ref in (rbuf, cbuf, vbuf):
                pltpu.make_async_copy(
                    rows_hbm.at[pl.ds(0, CHUNK)], ref.at[0], lsem
                ).wait()

        def _drain_streams(n):
            for _ in range(n):
                pltpu.make_async_copy(
                    vstage.at[0, pl.ds(0, 128)], acc.at[idxs.at[0, 0]], ssem
                ).wait()

        zh = [
            pltpu.async_copy(
                zeros, acc.at[pl.ds(s * TW_REG + i * ZCHUNK, ZCHUNK)], zsem
            )
            for i in range(NZCOPY)
        ]
        for h in zh:
            h.wait()
        plsc.subcore_barrier()

        _issue_loads(0, 0)

        def _chunk(k, _):
            par = lax.rem(k, 2)
            _drain_loads()

            @pl.when(k + 1 < NCHUNK)
            def _prefetch():
                _issue_loads(k + 1, 1 - par)

            @pl.when(k >= 2)
            def _older():
                _drain_streams(JROWS)

            for j in range(JROWS):
                def _vec(m, _):
                    o = pl.multiple_of(j * 128 + m * 16, 16)
                    sl = pl.ds(o, 16)
                    idx = cbuf[par, sl] * OUT_DIM + rbuf[par, sl] - base
                    valid = (idx >= 0) & (idx < REG)
                    idxs[par, j, pl.ds(pl.multiple_of(m * 16, 16), 16)] = (
                        jnp.where(valid, idx, sbase + (j * 128 + m * 16))
                    )
                    vstage[par, sl] = jnp.where(valid, vbuf[par, sl], 0.0)
                    return ()

                lax.fori_loop(0, 8, _vec, ())
                pltpu.async_copy(
                    vstage.at[par, pl.ds(j * 128, 128)],
                    acc.at[idxs.at[par, j]],
                    ssem,
                    add=True,
                )
            return ()

        lax.fori_loop(0, NCHUNK, _chunk, ())
        _drain_streams(2 * JROWS)
        plsc.subcore_barrier()

        src_full = acc.at[pl.ds(s * TW_REG, TW_REG)]
        dst_full = w_hbm.at[pl.ds(c * REG + s * TW_REG, TW_REG)]
        if rnd < NROUNDS - 1:
            pltpu.sync_copy(src_full, dst_full)
        else:
            @pl.when((c == 0) | (s < 13))
            def _full():
                pltpu.sync_copy(src_full, dst_full)

            @pl.when((c == 1) & (s == 13))
            def _partial():
                pltpu.sync_copy(
                    acc.at[pl.ds(13 * TW_REG, 57344)],
                    w_hbm.at[pl.ds(REG + 13 * TW_REG, 57344)],
                )

    return _scatter


_SCATTER_ROUNDS = [_make_scatter_round(r) for r in range(NROUNDS)]


BN = 1024


def _matmul_acc_body(x_ref, wt_ref, acc_ref, out_ref):
    out_ref[...] = acc_ref[...] + jnp.dot(
        x_ref[...],
        wt_ref[...].astype(jnp.bfloat16),
        preferred_element_type=jnp.float32,
    )


def _matmul_acc(x_r, wt_r, acc):
    kr = x_r.shape[1]
    return pl.pallas_call(
        _matmul_acc_body,
        grid=(OUT_DIM // BN,),
        in_specs=[
            pl.BlockSpec((BATCH, kr), lambda n: (0, 0)),
            pl.BlockSpec((kr, BN), lambda n: (0, n)),
            pl.BlockSpec((BATCH, BN), lambda n: (0, n)),
        ],
        out_specs=pl.BlockSpec((BATCH, BN), lambda n: (0, n)),
        out_shape=jax.ShapeDtypeStruct((BATCH, OUT_DIM), jnp.float32),
        input_output_aliases={2: 0},
        compiler_params=pltpu.CompilerParams(
            dimension_semantics=("arbitrary",),
        ),
    )(x_r, wt_r, acc)


def kernel(x, vals, b, rows, cols):
    pad = NNZ_PAD - rows.shape[0]
    t = jnp.arange(pad, dtype=jnp.int32)
    rows_p = jnp.concatenate([rows, (t * 37) % OUT_DIM])
    cols_p = jnp.concatenate([cols, t % IN_DIM])
    vals_p = jnp.concatenate([vals, jnp.zeros((pad,), jnp.float32)])

    xb = x.astype(jnp.bfloat16)
    acc = jnp.broadcast_to(b, (BATCH, OUT_DIM)) + jnp.zeros(
        (BATCH, OUT_DIM), jnp.float32
    )
    kr0 = 2 * ROWS_PER_REG
    for rnd in range(NROUNDS):
        wt_flat = _SCATTER_ROUNDS[rnd](rows_p, cols_p, vals_p)
        kr = wt_flat.shape[0] // OUT_DIM
        wt_r = wt_flat.reshape(kr, OUT_DIM)
        x_r = lax.slice_in_dim(xb, rnd * kr0, rnd * kr0 + kr, axis=1)
        acc = _matmul_acc(x_r, wt_r, acc)
    return acc
